# baseline (speedup 1.0000x reference)
import dataclasses

import jax
import jax.numpy as jnp
from jax import lax
from jax.experimental import pallas as pl
from jax.experimental.pallas import tpu as pltpu
from jax.experimental.pallas import tpu_sc as plsc

N = 10000
E = 160000
D = 256
H = 512
T = 4

R = 2600
NP = 4
NACC = 2688
PADDST = 10240
E2 = 163840
CH = 128
RPS = NACC // 16
RB = 200
NBLK = N // RB
RB2 = 400
NBLK2 = N // RB2

EPS_SUB = E2 // 16
PBUF = EPS_SUB + CH
TRASH = PBUF
PBUFF = PBUF + 16


def _table_body(x_ref, emb_ref, out_ref):
    xb = x_ref[...]
    for t in range(T):
        e = emb_ref[t:t + 1, :]
        v = jnp.maximum(xb + e, 0.0)

        @pl.when(pl.program_id(0) == 0)
        def _():
            out_ref[t] = v[:, :128]

        @pl.when(pl.program_id(0) == 1)
        def _():
            out_ref[t] = v[:, 128:]


def _build_table(x, emb8):
    out = pl.pallas_call(
        _table_body,
        grid=(2, NBLK2),
        in_specs=[
            pl.BlockSpec((RB2, D), lambda c, i: (i, 0)),
            pl.BlockSpec((T, D), lambda c, i: (0, 0)),
        ],
        out_specs=pl.BlockSpec((T, RB2, 128), lambda c, i: (c, i, 0)),
        out_shape=jax.ShapeDtypeStruct((2 * T, N, 128), jnp.float32),
    )(x, emb8)
    return out.reshape(2 * T * N, 128)


POOL = EPS_SUB + (NP + 1) * CH


def _sc_body(tabf, gidx_hbm, dst_hbm, out_hbm,
             gidx_v, dst_v, gbp, dbp, rows_a, rows_b, zbuf, acc_sh,
             sem_a, sem_b, sem_z, sem_sa, sem_sb):
    c = lax.axis_index("c")
    s = lax.axis_index("s")
    sl = pl.ds(s * RPS, RPS)

    pltpu.sync_copy(gidx_hbm.at[s], gidx_v)
    pltpu.sync_copy(dst_hbm.at[s], dst_v)

    z16 = jnp.zeros((16,), jnp.float32)

    @pl.loop(0, CH)
    def _(r):
        for k in range(128 // 16):
            zbuf[r, pl.ds(k * 16, 16)] = z16

    def count_body(i, carry):
        d = dst_v[pl.ds(i * 16, 16)]
        out = []
        for b in range(NP):
            lo, hi = b * R, min((b + 1) * R, N)
            m = jnp.logical_and(d >= lo, d < hi)
            out.append(carry[b] + jnp.sum(m.astype(jnp.int32), axis=0))
        return tuple(out)

    cnt = lax.fori_loop(0, EPS_SUB // 16, count_body, (0,) * NP)

    bases = [0]
    for b in range(NP - 1):
        bases.append(bases[b] + ((cnt[b] + CH - 1) // CH) * CH)

    iota16 = lax.iota(jnp.int32, 16)
    gfill = jnp.full((16,), c * T * N, jnp.int32)
    dfill = jnp.full((16,), R, jnp.int32)
    for b in range(NP):
        start = bases[b] + cnt[b]
        for k in range(CH // 16):
            pos = jnp.full((16,), start + k * 16, jnp.int32) + iota16
            plsc.store_scatter(gbp, [pos], gfill)
            plsc.store_scatter(dbp, [pos], dfill)

    coff = jnp.full((16,), c * T * N, jnp.int32)

    def part_body(i, carry):
        offs = carry
        d = dst_v[pl.ds(i * 16, 16)]
        g = gidx_v[pl.ds(i * 16, 16)] + coff
        trash = jnp.full((16,), POOL - 16, jnp.int32)
        new_offs = []
        for b in range(NP):
            lo, hi = b * R, min((b + 1) * R, N)
            m = jnp.logical_and(d >= lo, d < hi)
            cs = plsc.cumsum(m.astype(jnp.int32))
            pos = jnp.where(m, jnp.full((16,), offs[b] - 1, jnp.int32) + cs,
                            trash)
            plsc.store_scatter(gbp, [pos], g)
            plsc.store_scatter(dbp, [pos], d - lo)
            new_offs.append(offs[b] + jnp.sum(m.astype(jnp.int32), axis=0))
        return tuple(new_offs)

    lax.fori_loop(0, EPS_SUB // 16, part_body, tuple(bases))

    @pl.loop(0, NP)
    def _(p):
        nedges = cnt[NP - 1]
        base = bases[NP - 1]
        for b in range(NP - 1):
            nedges = jnp.where(p == b, cnt[b], nedges)
            base = jnp.where(p == b, bases[b], base)
        nch = jnp.maximum((nedges + CH - 1) // CH, 1)

        pltpu.async_copy(zbuf, acc_sh.at[pl.ds(s * RPS, CH)], sem_z).wait()
        pltpu.async_copy(zbuf.at[pl.ds(0, RPS - CH)],
                         acc_sh.at[pl.ds(s * RPS + CH, RPS - CH)],
                         sem_z).wait()
        plsc.subcore_barrier()

        def scatter(rows, j):
            for k in range(CH // 16):
                d16 = dbp[pl.ds(base + j * CH + k * 16, 16)]
                pltpu.sync_copy(rows.at[pl.ds(k * 16, 16)], acc_sh.at[d16],
                                add=True)

        def gather(j, buf, sem):
            return pltpu.make_async_copy(
                tabf.at[gbp.at[pl.ds(base + j * CH, CH)]], buf, sem)

        gather(0, rows_a, sem_a).start()

        @pl.loop(0, nch, step=2)
        def _(j):
            gather(j, rows_a, sem_a).wait()

            @pl.when(j + 1 < nch)
            def _():
                gather(j + 1, rows_b, sem_b).start()

            scatter(rows_a, j)

            @pl.when(j + 2 < nch)
            def _():
                gather(j + 2, rows_a, sem_a).start()

            @pl.when(j + 1 < nch)
            def _():
                gather(j + 1, rows_b, sem_b).wait()
                scatter(rows_b, j + 1)

        plsc.subcore_barrier()
        pltpu.sync_copy(acc_sh.at[sl], out_hbm.at[c, p, sl])
        plsc.subcore_barrier()


def _sc_aggregate(tabf, gidx, dst):
    mesh = plsc.VectorSubcoreMesh(core_axis_name="c", subcore_axis_name="s")
    cp = pltpu.CompilerParams()
    if "needs_layout_passes" in pltpu.CompilerParams.__dataclass_fields__:
        cp = dataclasses.replace(cp, needs_layout_passes=False)
    f = pl.kernel(
        _sc_body,
        out_type=jax.ShapeDtypeStruct((2, NP, NACC, 128), jnp.float32),
        compiler_params=cp,
        mesh=mesh,
        scratch_types=[
            pltpu.VMEM((EPS_SUB,), jnp.int32),
            pltpu.VMEM((EPS_SUB,), jnp.int32),
            pltpu.VMEM((POOL,), jnp.int32),
            pltpu.VMEM((POOL,), jnp.int32),
            pltpu.VMEM((CH, 128), jnp.float32),
            pltpu.VMEM((CH, 128), jnp.float32),
            pltpu.VMEM((CH, 128), jnp.float32),
            pltpu.VMEM_SHARED((NACC, 128), jnp.float32),
            pltpu.SemaphoreType.DMA,
            pltpu.SemaphoreType.DMA,
            pltpu.SemaphoreType.DMA,
            pltpu.SemaphoreType.DMA,
            pltpu.SemaphoreType.DMA,
        ],
    )
    return f(tabf, gidx, dst)


def _mlp1_body(x_ref, a0_ref, a1_ref, eps_ref, w1_ref, h1_ref, s_ref, q_ref):
    agg = jnp.concatenate([a0_ref[0, 0], a1_ref[0, 0]], axis=1)
    h = x_ref[...] * (1.0 + eps_ref[0, 0]) + agg
    h1 = jnp.dot(h.astype(jnp.bfloat16), w1_ref[...].astype(jnp.bfloat16),
                 preferred_element_type=jnp.float32)
    h1_ref[...] = h1
    s8 = jnp.sum(h1.reshape(RB // 8, 8, H), axis=0)
    q8 = jnp.sum((h1 * h1).reshape(RB // 8, 8, H), axis=0)

    @pl.when(pl.program_id(0) == 0)
    def _():
        s_ref[...] = s8
        q_ref[...] = q8

    @pl.when(pl.program_id(0) > 0)
    def _():
        s_ref[...] += s8
        q_ref[...] += q8


def _mlp1(x, aggr, epsr, W1):
    nb = R // RB
    return pl.pallas_call(
        _mlp1_body,
        grid=(NBLK,),
        in_specs=[
            pl.BlockSpec((RB, D), lambda i: (i, 0)),
            pl.BlockSpec((1, 1, RB, 128), lambda i: (0, i // nb, i % nb, 0)),
            pl.BlockSpec((1, 1, RB, 128), lambda i: (1, i // nb, i % nb, 0)),
            pl.BlockSpec(memory_space=pltpu.SMEM),
            pl.BlockSpec((D, H), lambda i: (0, 0)),
        ],
        out_specs=[
            pl.BlockSpec((RB, H), lambda i: (i, 0)),
            pl.BlockSpec((8, H), lambda i: (0, 0)),
            pl.BlockSpec((8, H), lambda i: (0, 0)),
        ],
        out_shape=[
            jax.ShapeDtypeStruct((N, H), jnp.float32),
            jax.ShapeDtypeStruct((8, H), jnp.float32),
            jax.ShapeDtypeStruct((8, H), jnp.float32),
        ],
    )(x, aggr, aggr, epsr, W1)


def _mlp2_body(h1_ref, s_ref, q_ref, p_ref, w2_ref, h2_ref, s2_ref, q2_ref):
    ssum = jnp.sum(s_ref[...], axis=0, keepdims=True)
    mean = ssum / N
    var = jnp.sum(q_ref[...], axis=0, keepdims=True) / N - mean * mean
    a = p_ref[0:1, :] * lax.rsqrt(var + 1e-5)
    cb = p_ref[1:2, :] - mean * a
    u = jnp.maximum(h1_ref[...] * a + cb, 0.0)
    h2 = jnp.dot(u.astype(jnp.bfloat16), w2_ref[...].astype(jnp.bfloat16),
                 preferred_element_type=jnp.float32)
    h2_ref[...] = h2
    s8 = jnp.sum(h2.reshape(RB2 // 8, 8, H), axis=0)
    q8 = jnp.sum((h2 * h2).reshape(RB2 // 8, 8, H), axis=0)

    @pl.when(pl.program_id(0) == 0)
    def _():
        s2_ref[...] = s8
        q2_ref[...] = q8

    @pl.when(pl.program_id(0) > 0)
    def _():
        s2_ref[...] += s8
        q2_ref[...] += q8


def _mlp2(h1, s1, q1, P1, W2):
    return pl.pallas_call(
        _mlp2_body,
        grid=(NBLK2,),
        in_specs=[
            pl.BlockSpec((RB2, H), lambda i: (i, 0)),
            pl.BlockSpec((8, H), lambda i: (0, 0)),
            pl.BlockSpec((8, H), lambda i: (0, 0)),
            pl.BlockSpec((8, H), lambda i: (0, 0)),
            pl.BlockSpec((H, H), lambda i: (0, 0)),
        ],
        out_specs=[
            pl.BlockSpec((RB2, H), lambda i: (i, 0)),
            pl.BlockSpec((8, H), lambda i: (0, 0)),
            pl.BlockSpec((8, H), lambda i: (0, 0)),
        ],
        out_shape=[
            jax.ShapeDtypeStruct((N, H), jnp.float32),
            jax.ShapeDtypeStruct((8, H), jnp.float32),
            jax.ShapeDtypeStruct((8, H), jnp.float32),
        ],
    )(h1, s1, q1, P1, W2)


def _mlp3_body(h2_ref, s_ref, q_ref, p_ref, w3_ref, b3_ref, x_ref, y_ref):
    ssum = jnp.sum(s_ref[...], axis=0, keepdims=True)
    mean = ssum / N
    var = jnp.sum(q_ref[...], axis=0, keepdims=True) / N - mean * mean
    a = p_ref[0:1, :] * lax.rsqrt(var + 1e-5)
    cb = p_ref[1:2, :] - mean * a
    u = jnp.maximum(h2_ref[...] * a + cb, 0.0)
    y = jnp.dot(u.astype(jnp.bfloat16), w3_ref[...].astype(jnp.bfloat16),
                 preferred_element_type=jnp.float32)
    y_ref[...] = y + b3_ref[0:1, :] + x_ref[...]


def _mlp3(h2, s2, q2, P2, W3, b3r, x):
    return pl.pallas_call(
        _mlp3_body,
        grid=(NBLK2,),
        in_specs=[
            pl.BlockSpec((RB2, H), lambda i: (i, 0)),
            pl.BlockSpec((8, H), lambda i: (0, 0)),
            pl.BlockSpec((8, H), lambda i: (0, 0)),
            pl.BlockSpec((8, H), lambda i: (0, 0)),
            pl.BlockSpec((H, D), lambda i: (0, 0)),
            pl.BlockSpec((8, D), lambda i: (0, 0)),
            pl.BlockSpec((RB2, D), lambda i: (i, 0)),
        ],
        out_specs=pl.BlockSpec((RB2, D), lambda i: (i, 0)),
        out_shape=jax.ShapeDtypeStruct((N, D), jnp.float32),
    )(h2, s2, q2, P2, W3, b3r, x)


def kernel(x, edge_index, edge_attr, edge_emb, eps, W1, g1, b1, W2, g2, b2,
           W3, b3):
    src = edge_index[0]
    dst = edge_index[1]

    tabf = _build_table(x, edge_emb)

    gidx = edge_attr * N + src
    gidx = jnp.concatenate([gidx, jnp.zeros((E2 - E,), jnp.int32)])
    dstp = jnp.concatenate([dst, jnp.full((E2 - E,), PADDST, jnp.int32)])
    gidx = gidx.reshape(16, EPS_SUB)
    dstp = dstp.reshape(16, EPS_SUB)
    aggr = _sc_aggregate(tabf, gidx, dstp)

    epsr = eps.reshape(1, 1)
    P1 = jnp.zeros((8, H), jnp.float32).at[0].set(g1).at[1].set(b1)
    P2 = jnp.zeros((8, H), jnp.float32).at[0].set(g2).at[1].set(b2)
    b3r = jnp.zeros((8, D), jnp.float32).at[0].set(b3)

    h1, s1, q1 = _mlp1(x, aggr, epsr, W1)
    h2, s2, q2 = _mlp2(h1, s1, q1, P1, W2)
    y = _mlp3(h2, s2, q2, P2, W3, b3r, x)
    return y


# speedup vs baseline: 1.0302x; 1.0302x over previous
import dataclasses

import jax
import jax.numpy as jnp
from jax import lax
from jax.experimental import pallas as pl
from jax.experimental.pallas import tpu as pltpu
from jax.experimental.pallas import tpu_sc as plsc

N = 10000
E = 160000
D = 256
H = 512
T = 4

R = 2600
NP = 4
NACC = 2688
PADDST = 10240
E2 = 163840
CH = 128
RPS = NACC // 16
RB = 200
NBLK = N // RB
RB2 = 400
NBLK2 = N // RB2

EPS_SUB = E2 // 16
PBUF = EPS_SUB + CH
TRASH = PBUF
PBUFF = PBUF + 16


def _table_body(x_ref, emb_ref, out_ref):
    xb = x_ref[...]
    for t in range(T):
        e = emb_ref[t:t + 1, :]
        v = jnp.maximum(xb + e, 0.0)

        @pl.when(pl.program_id(0) == 0)
        def _():
            out_ref[t] = v[:, :128]

        @pl.when(pl.program_id(0) == 1)
        def _():
            out_ref[t] = v[:, 128:]


def _build_table(x, emb8):
    out = pl.pallas_call(
        _table_body,
        grid=(2, NBLK2),
        in_specs=[
            pl.BlockSpec((RB2, D), lambda c, i: (i, 0)),
            pl.BlockSpec((T, D), lambda c, i: (0, 0)),
        ],
        out_specs=pl.BlockSpec((T, RB2, 128), lambda c, i: (c, i, 0)),
        out_shape=jax.ShapeDtypeStruct((2 * T, N, 128), jnp.float32),
    )(x, emb8)
    return out.reshape(2 * T * N, 128)


POOL = EPS_SUB + (NP + 1) * CH


def _sc_body(tabf, gidx_hbm, dst_hbm, out_hbm,
             gidx_v, dst_v, gbp, dbp, rows_a, rows_b, zbuf, acc_sh,
             sem_a, sem_b, sem_z, sem_sa, sem_sb):
    c = lax.axis_index("c")
    s = lax.axis_index("s")
    sl = pl.ds(s * RPS, RPS)

    pltpu.sync_copy(gidx_hbm.at[s], gidx_v)
    pltpu.sync_copy(dst_hbm.at[s], dst_v)

    z16 = jnp.zeros((16,), jnp.float32)

    @pl.loop(0, CH)
    def _(r):
        for k in range(128 // 16):
            zbuf[r, pl.ds(k * 16, 16)] = z16

    def count_body(i, carry):
        d = dst_v[pl.ds(i * 16, 16)]
        out = []
        for b in range(NP):
            lo, hi = b * R, min((b + 1) * R, N)
            m = jnp.logical_and(d >= lo, d < hi)
            out.append(carry[b] + jnp.sum(m.astype(jnp.int32), axis=0))
        return tuple(out)

    cnt = lax.fori_loop(0, EPS_SUB // 16, count_body, (0,) * NP)

    bases = [0]
    for b in range(NP - 1):
        bases.append(bases[b] + ((cnt[b] + CH - 1) // CH) * CH)

    iota16 = lax.iota(jnp.int32, 16)
    gfill = jnp.full((16,), c * T * N, jnp.int32)
    dfill = jnp.full((16,), R, jnp.int32)
    for b in range(NP):
        start = bases[b] + cnt[b]
        for k in range(CH // 16):
            pos = jnp.full((16,), start + k * 16, jnp.int32) + iota16
            plsc.store_scatter(gbp, [pos], gfill)
            plsc.store_scatter(dbp, [pos], dfill)

    coff = jnp.full((16,), c * T * N, jnp.int32)

    def part_body(i, carry):
        offs = carry
        d = dst_v[pl.ds(i * 16, 16)]
        g = gidx_v[pl.ds(i * 16, 16)] + coff
        trash = jnp.full((16,), POOL - 16, jnp.int32)
        new_offs = []
        for b in range(NP):
            lo, hi = b * R, min((b + 1) * R, N)
            m = jnp.logical_and(d >= lo, d < hi)
            cs = plsc.cumsum(m.astype(jnp.int32))
            pos = jnp.where(m, jnp.full((16,), offs[b] - 1, jnp.int32) + cs,
                            trash)
            plsc.store_scatter(gbp, [pos], g)
            plsc.store_scatter(dbp, [pos], d - lo)
            new_offs.append(offs[b] + jnp.sum(m.astype(jnp.int32), axis=0))
        return tuple(new_offs)

    lax.fori_loop(0, EPS_SUB // 16, part_body, tuple(bases))

    @pl.loop(0, NP)
    def _(p):
        nedges = cnt[NP - 1]
        base = bases[NP - 1]
        for b in range(NP - 1):
            nedges = jnp.where(p == b, cnt[b], nedges)
            base = jnp.where(p == b, bases[b], base)
        nch = jnp.maximum((nedges + CH - 1) // CH, 1)

        pltpu.async_copy(zbuf, acc_sh.at[pl.ds(s * RPS, CH)], sem_z).wait()
        pltpu.async_copy(zbuf.at[pl.ds(0, RPS - CH)],
                         acc_sh.at[pl.ds(s * RPS + CH, RPS - CH)],
                         sem_z).wait()
        plsc.subcore_barrier()

        def scatter(rows, j):
            for k in range(CH // 16):
                d16 = dbp[pl.ds(base + j * CH + k * 16, 16)]
                pltpu.sync_copy(rows.at[pl.ds(k * 16, 16)], acc_sh.at[d16],
                                add=True)

        def gather(j, buf, sem):
            return pltpu.make_async_copy(
                tabf.at[gbp.at[pl.ds(base + j * CH, CH)]], buf, sem)

        gather(0, rows_a, sem_a).start()

        @pl.loop(0, nch, step=2)
        def _(j):
            gather(j, rows_a, sem_a).wait()

            @pl.when(j + 1 < nch)
            def _():
                gather(j + 1, rows_b, sem_b).start()

            scatter(rows_a, j)

            @pl.when(j + 2 < nch)
            def _():
                gather(j + 2, rows_a, sem_a).start()

            @pl.when(j + 1 < nch)
            def _():
                gather(j + 1, rows_b, sem_b).wait()
                scatter(rows_b, j + 1)

        plsc.subcore_barrier()
        pltpu.sync_copy(acc_sh.at[sl], out_hbm.at[c, p, sl])
        plsc.subcore_barrier()


def _sc_aggregate(tabf, gidx, dst):
    mesh = plsc.VectorSubcoreMesh(core_axis_name="c", subcore_axis_name="s")
    cp = pltpu.CompilerParams()
    if "needs_layout_passes" in pltpu.CompilerParams.__dataclass_fields__:
        cp = dataclasses.replace(cp, needs_layout_passes=False)
    f = pl.kernel(
        _sc_body,
        out_type=jax.ShapeDtypeStruct((2, NP, NACC, 128), jnp.float32),
        compiler_params=cp,
        mesh=mesh,
        scratch_types=[
            pltpu.VMEM((EPS_SUB,), jnp.int32),
            pltpu.VMEM((EPS_SUB,), jnp.int32),
            pltpu.VMEM((POOL,), jnp.int32),
            pltpu.VMEM((POOL,), jnp.int32),
            pltpu.VMEM((CH, 128), jnp.float32),
            pltpu.VMEM((CH, 128), jnp.float32),
            pltpu.VMEM((CH, 128), jnp.float32),
            pltpu.VMEM_SHARED((NACC, 128), jnp.float32),
            pltpu.SemaphoreType.DMA,
            pltpu.SemaphoreType.DMA,
            pltpu.SemaphoreType.DMA,
            pltpu.SemaphoreType.DMA,
            pltpu.SemaphoreType.DMA,
        ],
    )
    return f(tabf, gidx, dst)


def _mlp1_body(x_ref, a0_ref, a1_ref, eps_ref, w1_ref, h1_ref, s_ref, q_ref):
    agg = jnp.concatenate([a0_ref[0, 0], a1_ref[0, 0]], axis=1)
    h = x_ref[...] * (1.0 + eps_ref[0, 0]) + agg
    h1 = jnp.dot(h.astype(jnp.bfloat16), w1_ref[...].astype(jnp.bfloat16),
                 preferred_element_type=jnp.float32)
    h1_ref[...] = h1.astype(jnp.bfloat16)
    s8 = jnp.sum(h1.reshape(RB // 8, 8, H), axis=0)
    q8 = jnp.sum((h1 * h1).reshape(RB // 8, 8, H), axis=0)

    @pl.when(pl.program_id(0) == 0)
    def _():
        s_ref[...] = s8
        q_ref[...] = q8

    @pl.when(pl.program_id(0) > 0)
    def _():
        s_ref[...] += s8
        q_ref[...] += q8


def _mlp1(x, aggr, epsr, W1):
    nb = R // RB
    return pl.pallas_call(
        _mlp1_body,
        grid=(NBLK,),
        in_specs=[
            pl.BlockSpec((RB, D), lambda i: (i, 0)),
            pl.BlockSpec((1, 1, RB, 128), lambda i: (0, i // nb, i % nb, 0)),
            pl.BlockSpec((1, 1, RB, 128), lambda i: (1, i // nb, i % nb, 0)),
            pl.BlockSpec(memory_space=pltpu.SMEM),
            pl.BlockSpec((D, H), lambda i: (0, 0)),
        ],
        out_specs=[
            pl.BlockSpec((RB, H), lambda i: (i, 0)),
            pl.BlockSpec((8, H), lambda i: (0, 0)),
            pl.BlockSpec((8, H), lambda i: (0, 0)),
        ],
        out_shape=[
            jax.ShapeDtypeStruct((N, H), jnp.bfloat16),
            jax.ShapeDtypeStruct((8, H), jnp.float32),
            jax.ShapeDtypeStruct((8, H), jnp.float32),
        ],
    )(x, aggr, aggr, epsr, W1)


def _mlp2_body(h1_ref, s_ref, q_ref, p_ref, w2_ref, h2_ref, s2_ref, q2_ref):
    ssum = jnp.sum(s_ref[...], axis=0, keepdims=True)
    mean = ssum / N
    var = jnp.sum(q_ref[...], axis=0, keepdims=True) / N - mean * mean
    a = p_ref[0:1, :] * lax.rsqrt(var + 1e-5)
    cb = p_ref[1:2, :] - mean * a
    u = jnp.maximum(h1_ref[...] * a + cb, 0.0)
    h2 = jnp.dot(u.astype(jnp.bfloat16), w2_ref[...].astype(jnp.bfloat16),
                 preferred_element_type=jnp.float32)
    h2_ref[...] = h2.astype(jnp.bfloat16)
    s8 = jnp.sum(h2.reshape(RB2 // 8, 8, H), axis=0)
    q8 = jnp.sum((h2 * h2).reshape(RB2 // 8, 8, H), axis=0)

    @pl.when(pl.program_id(0) == 0)
    def _():
        s2_ref[...] = s8
        q2_ref[...] = q8

    @pl.when(pl.program_id(0) > 0)
    def _():
        s2_ref[...] += s8
        q2_ref[...] += q8


def _mlp2(h1, s1, q1, P1, W2):
    return pl.pallas_call(
        _mlp2_body,
        grid=(NBLK2,),
        in_specs=[
            pl.BlockSpec((RB2, H), lambda i: (i, 0)),
            pl.BlockSpec((8, H), lambda i: (0, 0)),
            pl.BlockSpec((8, H), lambda i: (0, 0)),
            pl.BlockSpec((8, H), lambda i: (0, 0)),
            pl.BlockSpec((H, H), lambda i: (0, 0)),
        ],
        out_specs=[
            pl.BlockSpec((RB2, H), lambda i: (i, 0)),
            pl.BlockSpec((8, H), lambda i: (0, 0)),
            pl.BlockSpec((8, H), lambda i: (0, 0)),
        ],
        out_shape=[
            jax.ShapeDtypeStruct((N, H), jnp.bfloat16),
            jax.ShapeDtypeStruct((8, H), jnp.float32),
            jax.ShapeDtypeStruct((8, H), jnp.float32),
        ],
    )(h1, s1, q1, P1, W2)


def _mlp3_body(h2_ref, s_ref, q_ref, p_ref, w3_ref, b3_ref, x_ref, y_ref):
    ssum = jnp.sum(s_ref[...], axis=0, keepdims=True)
    mean = ssum / N
    var = jnp.sum(q_ref[...], axis=0, keepdims=True) / N - mean * mean
    a = p_ref[0:1, :] * lax.rsqrt(var + 1e-5)
    cb = p_ref[1:2, :] - mean * a
    u = jnp.maximum(h2_ref[...] * a + cb, 0.0)
    y = jnp.dot(u.astype(jnp.bfloat16), w3_ref[...].astype(jnp.bfloat16),
                 preferred_element_type=jnp.float32)
    y_ref[...] = y + b3_ref[0:1, :] + x_ref[...]


def _mlp3(h2, s2, q2, P2, W3, b3r, x):
    return pl.pallas_call(
        _mlp3_body,
        grid=(NBLK2,),
        in_specs=[
            pl.BlockSpec((RB2, H), lambda i: (i, 0)),
            pl.BlockSpec((8, H), lambda i: (0, 0)),
            pl.BlockSpec((8, H), lambda i: (0, 0)),
            pl.BlockSpec((8, H), lambda i: (0, 0)),
            pl.BlockSpec((H, D), lambda i: (0, 0)),
            pl.BlockSpec((8, D), lambda i: (0, 0)),
            pl.BlockSpec((RB2, D), lambda i: (i, 0)),
        ],
        out_specs=pl.BlockSpec((RB2, D), lambda i: (i, 0)),
        out_shape=jax.ShapeDtypeStruct((N, D), jnp.float32),
    )(h2, s2, q2, P2, W3, b3r, x)


def kernel(x, edge_index, edge_attr, edge_emb, eps, W1, g1, b1, W2, g2, b2,
           W3, b3):
    src = edge_index[0]
    dst = edge_index[1]

    tabf = _build_table(x, edge_emb)

    gidx = edge_attr * N + src
    gidx = jnp.concatenate([gidx, jnp.zeros((E2 - E,), jnp.int32)])
    dstp = jnp.concatenate([dst, jnp.full((E2 - E,), PADDST, jnp.int32)])
    gidx = gidx.reshape(16, EPS_SUB)
    dstp = dstp.reshape(16, EPS_SUB)
    aggr = _sc_aggregate(tabf, gidx, dstp)

    epsr = eps.reshape(1, 1)
    P1 = jnp.zeros((8, H), jnp.float32).at[0].set(g1).at[1].set(b1)
    P2 = jnp.zeros((8, H), jnp.float32).at[0].set(g2).at[1].set(b2)
    b3r = jnp.zeros((8, D), jnp.float32).at[0].set(b3)

    h1, s1, q1 = _mlp1(x, aggr, epsr, W1)
    h2, s2, q2 = _mlp2(h1, s1, q1, P1, W2)
    y = _mlp3(h2, s2, q2, P2, W3, b3r, x)
    return y
